# Initial kernel scaffold; baseline (speedup 1.0000x reference)
import jax
import jax.numpy as jnp
from jax.experimental import pallas as pl


def kernel(x, edge_types, t, means, temps, mul_w, bias_w):
    raise NotImplementedError("write your pallas kernel here")


# speedup vs baseline: 24.0119x; 24.0119x over previous
import functools

import jax
import jax.numpy as jnp
from jax import lax
from jax.experimental import pallas as pl
from jax.experimental.pallas import tpu as pltpu
from jax.experimental.pallas import tpu_sc as plsc

_LANES = 16


def _sc_affine(x_flat, et_flat, mul_flat, bias_flat):
    total = x_flat.shape[0]
    info = plsc.get_sparse_core_info()
    nw = info.num_cores * info.num_subcores
    chunk = total // nw
    assert chunk * nw == total and chunk % _LANES == 0 and chunk % 8 == 0
    table = mul_flat.shape[0]
    mesh = plsc.VectorSubcoreMesh(core_axis_name="c", subcore_axis_name="s")

    @functools.partial(
        pl.kernel,
        mesh=mesh,
        out_type=jax.ShapeDtypeStruct((total,), jnp.float32),
        compiler_params=pltpu.CompilerParams(needs_layout_passes=False),
        scratch_types=[
            pltpu.VMEM((chunk,), jnp.int32),
            pltpu.VMEM((chunk,), jnp.float32),
            pltpu.VMEM((chunk,), jnp.float32),
            pltpu.VMEM((table,), jnp.float32),
            pltpu.VMEM((table,), jnp.float32),
        ],
    )
    def sc_run(x_hbm, et_hbm, mul_hbm, bias_hbm, out_hbm,
               idx_v, x_v, out_v, mul_v, bias_v):
        wid = lax.axis_index("s") * info.num_cores + lax.axis_index("c")
        base = wid * chunk
        pltpu.sync_copy(et_hbm.at[pl.ds(base, chunk)], idx_v)
        pltpu.sync_copy(x_hbm.at[pl.ds(base, chunk)], x_v)
        pltpu.sync_copy(mul_hbm, mul_v)
        pltpu.sync_copy(bias_hbm, bias_v)

        def body(i, carry):
            sl = pl.ds(i * _LANES, _LANES)
            idx = idx_v[sl]
            m = plsc.load_gather(mul_v, [idx])
            b = plsc.load_gather(bias_v, [idx])
            out_v[sl] = m * x_v[sl] + b
            return carry

        lax.fori_loop(0, chunk // _LANES, body, 0)
        pltpu.sync_copy(out_v, out_hbm.at[pl.ds(base, chunk)])

    return sc_run(x_flat, et_flat, mul_flat, bias_flat)


def _tc_rbf(xs2, means, temps, bp):
    P, Q = xs2.shape
    K = means.shape[1]

    def body(xs_ref, mean_ref, temp_ref, out_ref):
        xsb = xs_ref[...]
        m = mean_ref[...][0]
        c = -jnp.abs(temp_ref[...][0])
        d = xsb[:, :, None] - m[None, None, :]
        out_ref[...] = jnp.exp(d * d * c[None, None, :])

    return pl.pallas_call(
        body,
        grid=(P // bp,),
        in_specs=[
            pl.BlockSpec((bp, Q), lambda i: (i, 0)),
            pl.BlockSpec((8, K), lambda i: (0, 0)),
            pl.BlockSpec((8, K), lambda i: (0, 0)),
        ],
        out_specs=pl.BlockSpec((bp, Q, K), lambda i: (i, 0, 0)),
        out_shape=jax.ShapeDtypeStruct((P, Q, K), jnp.float32),
    )(xs2, means, temps)


def kernel(x, edge_types, t, means, temps, mul_w, bias_w):
    B, N, _ = x.shape
    K = means.shape[1]
    total = B * N * N
    xf = x.reshape(total)
    ef = edge_types.reshape(total).astype(jnp.int32)
    xs = _sc_affine(xf, ef, mul_w.reshape(-1), bias_w.reshape(-1))
    Q = 128
    out = _tc_rbf(xs.reshape(total // Q, Q), means, temps, bp=8)
    return out.reshape(B, N, N, K)


# speedup vs baseline: 45.5223x; 1.8958x over previous
import functools

import jax
import jax.numpy as jnp
from jax import lax
from jax.experimental import pallas as pl
from jax.experimental.pallas import tpu as pltpu
from jax.experimental.pallas import tpu_sc as plsc

_LANES = 16


def _sc_affine(x_flat, et_flat, mul_flat, bias_flat):
    total = x_flat.shape[0]
    info = plsc.get_sparse_core_info()
    nw = info.num_cores * info.num_subcores
    chunk = total // nw
    assert chunk * nw == total and chunk % _LANES == 0 and chunk % 8 == 0
    table = mul_flat.shape[0]
    mesh = plsc.VectorSubcoreMesh(core_axis_name="c", subcore_axis_name="s")

    @functools.partial(
        pl.kernel,
        mesh=mesh,
        out_type=jax.ShapeDtypeStruct((total,), jnp.float32),
        compiler_params=pltpu.CompilerParams(needs_layout_passes=False),
        scratch_types=[
            pltpu.VMEM((chunk,), jnp.int32),
            pltpu.VMEM((chunk,), jnp.float32),
            pltpu.VMEM((chunk,), jnp.float32),
            pltpu.VMEM((table,), jnp.float32),
            pltpu.VMEM((table,), jnp.float32),
        ],
    )
    def sc_run(x_hbm, et_hbm, mul_hbm, bias_hbm, out_hbm,
               idx_v, x_v, out_v, mul_v, bias_v):
        wid = lax.axis_index("s") * info.num_cores + lax.axis_index("c")
        base = wid * chunk
        pltpu.sync_copy(et_hbm.at[pl.ds(base, chunk)], idx_v)
        pltpu.sync_copy(x_hbm.at[pl.ds(base, chunk)], x_v)
        pltpu.sync_copy(mul_hbm, mul_v)
        pltpu.sync_copy(bias_hbm, bias_v)

        def body(i, carry):
            sl = pl.ds(i * _LANES, _LANES)
            idx = idx_v[sl]
            m = plsc.load_gather(mul_v, [idx])
            b = plsc.load_gather(bias_v, [idx])
            out_v[sl] = m * x_v[sl] + b
            return carry

        lax.fori_loop(0, chunk // _LANES, body, 0)
        pltpu.sync_copy(out_v, out_hbm.at[pl.ds(base, chunk)])

    return sc_run(x_flat, et_flat, mul_flat, bias_flat)


def _tc_rbf(xs2, means, temps, bp):
    P, Q = xs2.shape
    K = means.shape[1]

    def body(xs_ref, mean_ref, temp_ref, out_ref):
        xsb = xs_ref[...]
        m = mean_ref[...][0]
        c = jnp.abs(temp_ref[...][0]) * (-1.4426950408889634)
        d = xsb[:, :, None] - m[None, None, :]
        out_ref[...] = jnp.exp2(d * d * c[None, None, :])

    return pl.pallas_call(
        body,
        grid=(P // bp,),
        in_specs=[
            pl.BlockSpec((bp, Q), lambda i: (i, 0)),
            pl.BlockSpec((8, K), lambda i: (0, 0)),
            pl.BlockSpec((8, K), lambda i: (0, 0)),
        ],
        out_specs=pl.BlockSpec((bp, Q, K), lambda i: (i, 0, 0)),
        out_shape=jax.ShapeDtypeStruct((P, Q, K), jnp.float32),
    )(xs2, means, temps)


def kernel(x, edge_types, t, means, temps, mul_w, bias_w):
    B, N, _ = x.shape
    K = means.shape[1]
    total = B * N * N
    xf = x.reshape(total)
    ef = edge_types.reshape(total).astype(jnp.int32)
    xs = _sc_affine(xf, ef, mul_w.reshape(-1), bias_w.reshape(-1))
    Q = 128
    out = _tc_rbf(xs.reshape(total // Q, Q), means, temps, bp=32)
    return out.reshape(B, N, N, K)


# speedup vs baseline: 53.8779x; 1.1835x over previous
import functools

import jax
import jax.numpy as jnp
from jax import lax
from jax.experimental import pallas as pl
from jax.experimental.pallas import tpu as pltpu
from jax.experimental.pallas import tpu_sc as plsc

_LANES = 16


def _sc_affine(x_flat, et_flat, mul_flat, bias_flat):
    total = x_flat.shape[0]
    info = plsc.get_sparse_core_info()
    nw = info.num_cores * info.num_subcores
    chunk = total // nw
    assert chunk * nw == total and chunk % _LANES == 0 and chunk % 8 == 0
    table = mul_flat.shape[0]
    mesh = plsc.VectorSubcoreMesh(core_axis_name="c", subcore_axis_name="s")

    @functools.partial(
        pl.kernel,
        mesh=mesh,
        out_type=jax.ShapeDtypeStruct((total,), jnp.float32),
        compiler_params=pltpu.CompilerParams(needs_layout_passes=False),
        scratch_types=[
            pltpu.VMEM((chunk,), jnp.int32),
            pltpu.VMEM((chunk,), jnp.float32),
            pltpu.VMEM((chunk,), jnp.float32),
            pltpu.VMEM((table,), jnp.float32),
            pltpu.VMEM((table,), jnp.float32),
        ],
    )
    def sc_run(x_hbm, et_hbm, mul_hbm, bias_hbm, out_hbm,
               idx_v, x_v, out_v, mul_v, bias_v):
        wid = lax.axis_index("s") * info.num_cores + lax.axis_index("c")
        base = wid * chunk
        pltpu.sync_copy(et_hbm.at[pl.ds(base, chunk)], idx_v)
        pltpu.sync_copy(x_hbm.at[pl.ds(base, chunk)], x_v)
        pltpu.sync_copy(mul_hbm, mul_v)
        pltpu.sync_copy(bias_hbm, bias_v)

        def body(i, carry):
            sl = pl.ds(i * _LANES, _LANES)
            idx = idx_v[sl]
            m = plsc.load_gather(mul_v, [idx])
            b = plsc.load_gather(bias_v, [idx])
            out_v[sl] = m * x_v[sl] + b
            return carry

        lax.fori_loop(0, chunk // _LANES, body, 0)
        pltpu.sync_copy(out_v, out_hbm.at[pl.ds(base, chunk)])

    return sc_run(x_flat, et_flat, mul_flat, bias_flat)


def _tc_rbf(xs2, means, temps, bp):
    P, Q = xs2.shape
    K = means.shape[1]

    def body(xs_ref, mean_ref, temp_ref, out_ref):
        xsb = xs_ref[...]
        m = mean_ref[...][0]
        c = jnp.abs(temp_ref[...][0]) * (-1.4426950408889634)
        d = xsb[:, :, None] - m[None, None, :]
        out_ref[...] = jnp.exp2(d * d * c[None, None, :])

    return pl.pallas_call(
        body,
        grid=(P // bp,),
        in_specs=[
            pl.BlockSpec((bp, Q), lambda i: (i, 0)),
            pl.BlockSpec((8, K), lambda i: (0, 0)),
            pl.BlockSpec((8, K), lambda i: (0, 0)),
        ],
        out_specs=pl.BlockSpec((bp, Q, K), lambda i: (i, 0, 0)),
        out_shape=jax.ShapeDtypeStruct((P, Q, K), jnp.float32),
    )(xs2, means, temps)


def kernel(x, edge_types, t, means, temps, mul_w, bias_w):
    B, N, _ = x.shape
    K = means.shape[1]
    total = B * N * N
    xf = x.reshape(total)
    ef = edge_types.reshape(total).astype(jnp.int32)
    xs = _sc_affine(xf, ef, mul_w.reshape(-1), bias_w.reshape(-1))
    Q = 128
    out = _tc_rbf(xs.reshape(total // Q, Q), means, temps, bp=64)
    return out.reshape(B, N, N, K)


# speedup vs baseline: 57.0442x; 1.0588x over previous
import functools

import jax
import jax.numpy as jnp
from jax import lax
from jax.experimental import pallas as pl
from jax.experimental.pallas import tpu as pltpu
from jax.experimental.pallas import tpu_sc as plsc

_LANES = 16


def _sc_affine(x_flat, et_flat, mul_flat, bias_flat):
    total = x_flat.shape[0]
    info = plsc.get_sparse_core_info()
    nw = info.num_cores * info.num_subcores
    chunk = total // nw
    assert chunk * nw == total and chunk % _LANES == 0 and chunk % 8 == 0
    table = mul_flat.shape[0]
    mesh = plsc.VectorSubcoreMesh(core_axis_name="c", subcore_axis_name="s")

    @functools.partial(
        pl.kernel,
        mesh=mesh,
        out_type=jax.ShapeDtypeStruct((total,), jnp.float32),
        compiler_params=pltpu.CompilerParams(needs_layout_passes=False),
        scratch_types=[
            pltpu.VMEM((chunk,), jnp.int32),
            pltpu.VMEM((chunk,), jnp.float32),
            pltpu.VMEM((chunk,), jnp.float32),
            pltpu.VMEM((table,), jnp.float32),
            pltpu.VMEM((table,), jnp.float32),
        ],
    )
    def sc_run(x_hbm, et_hbm, mul_hbm, bias_hbm, out_hbm,
               idx_v, x_v, out_v, mul_v, bias_v):
        wid = lax.axis_index("s") * info.num_cores + lax.axis_index("c")
        base = wid * chunk
        pltpu.sync_copy(et_hbm.at[pl.ds(base, chunk)], idx_v)
        pltpu.sync_copy(x_hbm.at[pl.ds(base, chunk)], x_v)
        pltpu.sync_copy(mul_hbm, mul_v)
        pltpu.sync_copy(bias_hbm, bias_v)

        def body(i, carry):
            sl = pl.ds(i * _LANES, _LANES)
            idx = idx_v[sl]
            m = plsc.load_gather(mul_v, [idx])
            b = plsc.load_gather(bias_v, [idx])
            out_v[sl] = m * x_v[sl] + b
            return carry

        lax.fori_loop(0, chunk // _LANES, body, 0)
        pltpu.sync_copy(out_v, out_hbm.at[pl.ds(base, chunk)])

    return sc_run(x_flat, et_flat, mul_flat, bias_flat)


def _tc_rbf(xs2, means, temps, bp):
    P, Q = xs2.shape
    K = means.shape[1]

    def body(xs_ref, mean_ref, temp_ref, out_ref):
        xsb = xs_ref[...]
        m = mean_ref[...][0]
        c = jnp.abs(temp_ref[...][0]) * (-1.4426950408889634)
        d = xsb[:, :, None] - m[None, None, :]
        out_ref[...] = jnp.exp2(d * d * c[None, None, :])

    return pl.pallas_call(
        body,
        grid=(P // bp,),
        in_specs=[
            pl.BlockSpec((bp, Q), lambda i: (i, 0)),
            pl.BlockSpec((8, K), lambda i: (0, 0)),
            pl.BlockSpec((8, K), lambda i: (0, 0)),
        ],
        out_specs=pl.BlockSpec((bp, Q, K), lambda i: (i, 0, 0)),
        out_shape=jax.ShapeDtypeStruct((P, Q, K), jnp.float32),
    )(xs2, means, temps)


def kernel(x, edge_types, t, means, temps, mul_w, bias_w):
    B, N, _ = x.shape
    K = means.shape[1]
    total = B * N * N
    xf = x.reshape(total)
    ef = edge_types.reshape(total).astype(jnp.int32)
    xs = _sc_affine(xf, ef, mul_w.reshape(-1), bias_w.reshape(-1))
    Q = 128
    out = _tc_rbf(xs.reshape(total // Q, Q), means, temps, bp=128)
    return out.reshape(B, N, N, K)


# speedup vs baseline: 82.4160x; 1.4448x over previous
import jax
import jax.numpy as jnp
from jax.experimental import pallas as pl


def _tc_rbf_gather(x2, et2, means, temps, mul_t, bias_t, bp):
    P, Q = x2.shape
    K = means.shape[1]

    def body(x_ref, et_ref, mean_ref, temp_ref, mul_ref, bias_ref, out_ref):
        xb = x_ref[...]
        et = et_ref[...]
        m = mean_ref[...][0]
        c = jnp.abs(temp_ref[...][0]) * (-1.4426950408889634)
        low = et & 127
        hi = et >> 7
        mul_v = jnp.zeros_like(xb)
        bias_v = jnp.zeros_like(xb)
        for ch in range(8):
            mrow = mul_ref[...][ch]
            brow = bias_ref[...][ch]
            mg = jnp.take_along_axis(
                jnp.broadcast_to(mrow[None, :], (xb.shape[0], 128)), low, axis=1)
            bg = jnp.take_along_axis(
                jnp.broadcast_to(brow[None, :], (xb.shape[0], 128)), low, axis=1)
            sel = hi == ch
            mul_v = jnp.where(sel, mg, mul_v)
            bias_v = jnp.where(sel, bg, bias_v)
        xs = mul_v * xb + bias_v
        d = xs[:, :, None] - m[None, None, :]
        out_ref[...] = jnp.exp2(d * d * c[None, None, :])

    return pl.pallas_call(
        body,
        grid=(P // bp,),
        in_specs=[
            pl.BlockSpec((bp, Q), lambda i: (i, 0)),
            pl.BlockSpec((bp, Q), lambda i: (i, 0)),
            pl.BlockSpec((8, K), lambda i: (0, 0)),
            pl.BlockSpec((8, K), lambda i: (0, 0)),
            pl.BlockSpec((8, 128), lambda i: (0, 0)),
            pl.BlockSpec((8, 128), lambda i: (0, 0)),
        ],
        out_specs=pl.BlockSpec((bp, Q, K), lambda i: (i, 0, 0)),
        out_shape=jax.ShapeDtypeStruct((P, Q, K), jnp.float32),
    )(x2, et2, means, temps, mul_t, bias_t)


def kernel(x, edge_types, t, means, temps, mul_w, bias_w):
    B, N, _ = x.shape
    K = means.shape[1]
    total = B * N * N
    Q = 128
    x2 = x.reshape(B * N, N)
    et2 = edge_types.reshape(B * N, N).astype(jnp.int32)
    mul_t = mul_w.reshape(8, 128)
    bias_t = bias_w.reshape(8, 128)
    out = _tc_rbf_gather(x2, et2, means, temps, mul_t, bias_t, bp=64)
    return out.reshape(B, N, N, K)
